# baseline (speedup 1.0000x reference)
import jax
import jax.numpy as jnp
from jax import lax
from jax.experimental import pallas as pl
from jax.experimental.pallas import tpu as pltpu
from jax.experimental.pallas import tpu_sc as plsc

N = 10000
E = 320000
D = 128

NC = 2
NS = 16
NW = NC * NS
CHUNK = 128
TCH = E // CHUNK
NCH = TCH // NW
R3 = 3
R4 = 4
ZCH = 80
NZC = N // ZCH


def _sc_body(x_hbm, ei_hbm, w_hbm, out_hbm, rows, idx, wbuf, acc, sems):
  c = lax.axis_index("c")
  s = lax.axis_index("s")
  wid = s * NC + c
  nbase = wid * NCH + jnp.minimum(wid, TCH - NCH * NW)
  sem_i, sem_g, sem_s = sems

  def _eoff(m):
    return pl.multiple_of((nbase + m) * CHUNK, CHUNK)

  def _stage(m, r4, r3):
    off = _eoff(m)
    pltpu.async_copy(ei_hbm.at[:, pl.ds(off, CHUNK)], idx[r4], sem_i[r4])
    pltpu.async_copy(w_hbm.at[pl.ds(off, CHUNK)], wbuf[r3], sem_i[r4])

  def _wait_stage(r4, r3):
    pltpu.make_async_copy(ei_hbm.at[:, pl.ds(0, CHUNK)], idx[r4],
                          sem_i[r4]).wait()
    pltpu.make_async_copy(w_hbm.at[pl.ds(0, CHUNK)], wbuf[r3],
                          sem_i[r4]).wait()

  def _gather(r4, r3):
    pltpu.async_copy(x_hbm.at[idx[r4].at[0]], rows[r3], sem_g[r3])

  def _wait_gather(r4, r3):
    pltpu.make_async_copy(x_hbm.at[idx[r4].at[0]], rows[r3],
                          sem_g[r3]).wait()

  def _scatter(r4, r3):
    pltpu.async_copy(rows[r3], acc.at[idx[r4].at[1]], sem_s[r3], add=True)

  def _wait_scatter(r4, r3):
    pltpu.make_async_copy(rows[r3], acc.at[idx[r4].at[1]], sem_s[r3]).wait()

  def _scale(r4, r3):
    del r4

    @plsc.parallel_loop(0, CHUNK, unroll=4)
    def _(e):
      wb = plsc.load_gather(wbuf[r3], [jnp.full((16,), e, jnp.int32)])
      for j in range(D // 16):
        rows[r3][e, pl.ds(j * 16, 16)] = rows[r3][e, pl.ds(j * 16, 16)] * wb

  zvec = jnp.zeros((16,), jnp.float32)
  zview = rows[0].at[pl.ds(0, ZCH)]

  def _zero_row(i):
    for j in range(D // 16):
      rows[0][i, pl.ds(j * 16, 16)] = zvec
  pl.loop(0, ZCH)(_zero_row)

  def _zero_wait(r):
    pltpu.make_async_copy(zview, acc.at[pl.ds(0, ZCH)], sem_s[r]).wait()

  def _zq(k0):
    for rr in range(R3):
      k = k0 + rr * NS

      @pl.when(k < NZC)
      def _():
        @pl.when(k >= s + NS * R3)
        def _():
          _zero_wait(rr)
        pltpu.async_copy(zview, acc.at[pl.ds(k * ZCH, ZCH)], sem_s[rr])
  pl.loop(s, NZC, step=NS * R3)(_zq)
  for rr in range(R3):
    _zero_wait(rr)

  plsc.subcore_barrier()

  _stage(0, 0, 0)
  _stage(1, 1, 1)
  _wait_stage(0, 0)
  _gather(0, 0)

  def _body(m, d4, d3):
    @pl.when(m >= 2)
    def _():
      _wait_scatter((d4 + 2) % R4, (d3 + 1) % R3)

    @pl.when(m + 2 < NCH)
    def _():
      _stage(m + 2, (d4 + 2) % R4, (d3 + 2) % R3)

    @pl.when(m + 1 < NCH)
    def _():
      _wait_stage((d4 + 1) % R4, (d3 + 1) % R3)
      _gather((d4 + 1) % R4, (d3 + 1) % R3)

    _wait_gather(d4, d3)
    _scale(d4, d3)
    _scatter(d4, d3)

  def _block(m0):
    for dd in range(12):
      m = m0 + dd

      @pl.when(m < NCH)
      def _():
        _body(m, dd % R4, dd % R3)
  pl.loop(0, NCH, step=12)(_block)

  _wait_scatter((NCH - 2) % R4, (NCH - 2) % R3)
  _wait_scatter((NCH - 1) % R4, (NCH - 1) % R3)

  @pl.when(wid < TCH - NCH * NW)
  def _():
    _stage(NCH, 0, 0)
    _wait_stage(0, 0)
    _gather(0, 0)
    _wait_gather(0, 0)
    _scale(0, 0)
    _scatter(0, 0)
    _wait_scatter(0, 0)

  plsc.subcore_barrier()

  def _wo_wait(rr):
    pltpu.make_async_copy(zview, out_hbm.at[c, pl.ds(0, ZCH)],
                          sem_s[rr]).wait()

  def _wq(k0):
    for rr in range(R3):
      k = k0 + rr * NS

      @pl.when(k < NZC)
      def _():
        @pl.when(k >= s + NS * R3)
        def _():
          _wo_wait(rr)
        r0 = k * ZCH
        view = rows[rr].at[pl.ds(0, ZCH)]
        pltpu.sync_copy(acc.at[pl.ds(r0, ZCH)], view)
        pltpu.async_copy(view, out_hbm.at[c, pl.ds(r0, ZCH)], sem_s[rr])
  pl.loop(s, NZC, step=NS * R3)(_wq)
  for rr in range(R3):
    _wo_wait(rr)


_sc_scatter = pl.kernel(
    _sc_body,
    out_type=jax.ShapeDtypeStruct((NC, N, D), jnp.float32),
    mesh=plsc.VectorSubcoreMesh(core_axis_name="c", subcore_axis_name="s"),
    scratch_types=[
        tuple(pltpu.VMEM((CHUNK, D), jnp.float32) for _ in range(R3)),
        tuple(pltpu.VMEM((2, CHUNK), jnp.int32) for _ in range(R4)),
        tuple(pltpu.VMEM((CHUNK,), jnp.float32) for _ in range(R3)),
        pltpu.VMEM_SHARED((N, D), jnp.float32),
        (
            tuple(pltpu.SemaphoreType.DMA for _ in range(R4)),
            tuple(pltpu.SemaphoreType.DMA for _ in range(R3)),
            tuple(pltpu.SemaphoreType.DMA for _ in range(R3)),
        ),
    ],
    compiler_params=pltpu.CompilerParams(needs_layout_passes=False),
)


BLK = 10000


def _dense_body(x_ref, p_ref, wrel_ref, brel_ref, wroot_ref,
                wcat_ref, bcat_ref, mu_ref, ls_ref):
  agg = p_ref[0] + p_ref[1]
  h = jnp.dot(agg, wrel_ref[...], preferred_element_type=jnp.float32)
  h += jnp.dot(x_ref[...], wroot_ref[...], preferred_element_type=jnp.float32)
  h = jnp.maximum(h + brel_ref[...], 0.0)
  t = jnp.dot(h, wcat_ref[...], preferred_element_type=jnp.float32)
  t = jnp.maximum(t + bcat_ref[...], 0.0)
  mu_ref[...] = t[:, :D]
  ls_ref[...] = t[:, D:]


def _row_blk(i):
  return (i, 0)


def _p_blk(i):
  return (0, i, 0)


def _full(i):
  return (0, 0)


_dense = pl.pallas_call(
    _dense_body,
    grid=(N // BLK,),
    in_specs=[
        pl.BlockSpec((BLK, D), _row_blk),
        pl.BlockSpec((NC, BLK, D), _p_blk),
        pl.BlockSpec((D, D), _full),
        pl.BlockSpec((1, D), _full),
        pl.BlockSpec((D, D), _full),
        pl.BlockSpec((D, 2 * D), _full),
        pl.BlockSpec((1, 2 * D), _full),
    ],
    out_specs=[
        pl.BlockSpec((BLK, D), _row_blk),
        pl.BlockSpec((BLK, D), _row_blk),
    ],
    out_shape=[
        jax.ShapeDtypeStruct((N, D), jnp.float32),
        jax.ShapeDtypeStruct((N, D), jnp.float32),
    ],
)


@jax.jit
def kernel(x, edge_index, edge_weight, W_rel, b_rel, W_root, W_mu, b_mu,
           W_std, b_std):
  partials = _sc_scatter(x, edge_index.astype(jnp.int32), edge_weight)
  wcat = jnp.concatenate([W_mu.T, W_std.T], axis=1)
  bcat = jnp.concatenate([b_mu, b_std]).reshape(1, 2 * D)
  mu, ls = _dense(
      x, partials,
      W_rel.T, b_rel.reshape(1, D), W_root.T, wcat, bcat,
  )
  return (mu, ls)


# speedup vs baseline: 1.0194x; 1.0194x over previous
import jax
import jax.numpy as jnp
from jax import lax
from jax.experimental import pallas as pl
from jax.experimental.pallas import tpu as pltpu
from jax.experimental.pallas import tpu_sc as plsc

N = 10000
E = 320000
D = 128

NC = 2
NS = 16
NW = NC * NS
CHUNK = 128
TCH = E // CHUNK
NCH = TCH // NW
R3 = 3
R4 = 4
ZCH = 80
NZC = N // ZCH


def _sc_body(x_hbm, ei_hbm, w_hbm, out_hbm, rows, idx, wbuf, acc, sems):
  c = lax.axis_index("c")
  s = lax.axis_index("s")
  wid = s * NC + c
  nbase = wid * NCH + jnp.minimum(wid, TCH - NCH * NW)
  sem_i, sem_g, sem_s = sems

  def _eoff(m):
    return pl.multiple_of((nbase + m) * CHUNK, CHUNK)

  def _stage(m, r4, r3):
    off = _eoff(m)
    pltpu.async_copy(ei_hbm.at[:, pl.ds(off, CHUNK)], idx[r4], sem_i[r4])
    pltpu.async_copy(w_hbm.at[pl.ds(off, CHUNK)], wbuf[r3], sem_i[r4])

  def _wait_stage(r4, r3):
    pltpu.make_async_copy(ei_hbm.at[:, pl.ds(0, CHUNK)], idx[r4],
                          sem_i[r4]).wait()
    pltpu.make_async_copy(w_hbm.at[pl.ds(0, CHUNK)], wbuf[r3],
                          sem_i[r4]).wait()

  def _gather(r4, r3):
    pltpu.async_copy(x_hbm.at[idx[r4].at[0]], rows[r3], sem_g[r3])

  def _wait_gather(r4, r3):
    pltpu.make_async_copy(x_hbm.at[idx[r4].at[0]], rows[r3],
                          sem_g[r3]).wait()

  def _scatter(r4, r3):
    pltpu.async_copy(rows[r3], acc.at[idx[r4].at[1]], sem_s[r3], add=True)

  def _wait_scatter(r4, r3):
    pltpu.make_async_copy(rows[r3], acc.at[idx[r4].at[1]], sem_s[r3]).wait()

  def _scale(r4, r3):
    del r4

    @plsc.parallel_loop(0, CHUNK, unroll=4)
    def _(e):
      wb = plsc.load_gather(wbuf[r3], [jnp.full((16,), e, jnp.int32)])
      for j in range(D // 16):
        rows[r3][e, pl.ds(j * 16, 16)] = rows[r3][e, pl.ds(j * 16, 16)] * wb

  zvec = jnp.zeros((16,), jnp.float32)
  zview = rows[0].at[pl.ds(0, ZCH)]

  def _zero_row(i):
    for j in range(D // 16):
      rows[0][i, pl.ds(j * 16, 16)] = zvec
  pl.loop(0, ZCH)(_zero_row)

  def _zero_wait(r):
    pltpu.make_async_copy(zview, acc.at[pl.ds(0, ZCH)], sem_s[r]).wait()

  def _zq(k0):
    for rr in range(R3):
      k = k0 + rr * NS

      @pl.when(k < NZC)
      def _():
        @pl.when(k >= s + NS * R3)
        def _():
          _zero_wait(rr)
        pltpu.async_copy(zview, acc.at[pl.ds(k * ZCH, ZCH)], sem_s[rr])
  pl.loop(s, NZC, step=NS * R3)(_zq)
  for rr in range(R3):
    _zero_wait(rr)

  plsc.subcore_barrier()

  _stage(0, 0, 0)
  _stage(1, 1, 1)
  _wait_stage(0, 0)
  _gather(0, 0)

  def _body(m, d4, d3):
    @pl.when(m >= 2)
    def _():
      _wait_scatter((d4 + 2) % R4, (d3 + 1) % R3)

    @pl.when(m + 2 < NCH)
    def _():
      _stage(m + 2, (d4 + 2) % R4, (d3 + 2) % R3)

    @pl.when(m + 1 < NCH)
    def _():
      _wait_stage((d4 + 1) % R4, (d3 + 1) % R3)
      _gather((d4 + 1) % R4, (d3 + 1) % R3)

    _wait_gather(d4, d3)
    _scale(d4, d3)
    _scatter(d4, d3)

  def _block(m0):
    for dd in range(12):
      m = m0 + dd

      @pl.when(m < NCH)
      def _():
        _body(m, dd % R4, dd % R3)
  pl.loop(0, NCH, step=12)(_block)

  _wait_scatter((NCH - 2) % R4, (NCH - 2) % R3)
  _wait_scatter((NCH - 1) % R4, (NCH - 1) % R3)

  @pl.when(wid < TCH - NCH * NW)
  def _():
    _stage(NCH, 0, 0)
    _wait_stage(0, 0)
    _gather(0, 0)
    _wait_gather(0, 0)
    _scale(0, 0)
    _scatter(0, 0)
    _wait_scatter(0, 0)

  plsc.subcore_barrier()

  def _wo_wait(rr):
    pltpu.make_async_copy(zview, out_hbm.at[c, pl.ds(0, ZCH)],
                          sem_s[rr]).wait()

  def _wq(k0):
    for rr in range(R3):
      k = k0 + rr * NS

      @pl.when(k < NZC)
      def _():
        @pl.when(k >= s + NS * R3)
        def _():
          _wo_wait(rr)
        r0 = k * ZCH
        view = rows[rr].at[pl.ds(0, ZCH)]
        pltpu.sync_copy(acc.at[pl.ds(r0, ZCH)], view)
        pltpu.async_copy(view, out_hbm.at[c, pl.ds(r0, ZCH)], sem_s[rr])
  pl.loop(s, NZC, step=NS * R3)(_wq)
  for rr in range(R3):
    _wo_wait(rr)


_sc_scatter = pl.kernel(
    _sc_body,
    out_type=jax.ShapeDtypeStruct((NC, N, D), jnp.float32),
    mesh=plsc.VectorSubcoreMesh(core_axis_name="c", subcore_axis_name="s"),
    scratch_types=[
        tuple(pltpu.VMEM((CHUNK, D), jnp.float32) for _ in range(R3)),
        tuple(pltpu.VMEM((2, CHUNK), jnp.int32) for _ in range(R4)),
        tuple(pltpu.VMEM((CHUNK,), jnp.float32) for _ in range(R3)),
        pltpu.VMEM_SHARED((N, D), jnp.float32),
        (
            tuple(pltpu.SemaphoreType.DMA for _ in range(R4)),
            tuple(pltpu.SemaphoreType.DMA for _ in range(R3)),
            tuple(pltpu.SemaphoreType.DMA for _ in range(R3)),
        ),
    ],
    compiler_params=pltpu.CompilerParams(needs_layout_passes=False),
)


BLK = 5000


def _dense_body(x_ref, p_ref, wrel_ref, brel_ref, wroot_ref,
                wcat_ref, bcat_ref, mu_ref, ls_ref):
  agg = p_ref[0] + p_ref[1]
  h = jnp.dot(agg, wrel_ref[...], preferred_element_type=jnp.float32)
  h += jnp.dot(x_ref[...], wroot_ref[...], preferred_element_type=jnp.float32)
  h = jnp.maximum(h + brel_ref[...], 0.0)
  t = jnp.dot(h, wcat_ref[...], preferred_element_type=jnp.float32)
  t = jnp.maximum(t + bcat_ref[...], 0.0)
  mu_ref[...] = t[:, :D]
  ls_ref[...] = t[:, D:]


def _row_blk(i):
  return (i, 0)


def _p_blk(i):
  return (0, i, 0)


def _full(i):
  return (0, 0)


_dense = pl.pallas_call(
    _dense_body,
    grid=(N // BLK,),
    in_specs=[
        pl.BlockSpec((BLK, D), _row_blk),
        pl.BlockSpec((NC, BLK, D), _p_blk),
        pl.BlockSpec((D, D), _full),
        pl.BlockSpec((1, D), _full),
        pl.BlockSpec((D, D), _full),
        pl.BlockSpec((D, 2 * D), _full),
        pl.BlockSpec((1, 2 * D), _full),
    ],
    out_specs=[
        pl.BlockSpec((BLK, D), _row_blk),
        pl.BlockSpec((BLK, D), _row_blk),
    ],
    out_shape=[
        jax.ShapeDtypeStruct((N, D), jnp.float32),
        jax.ShapeDtypeStruct((N, D), jnp.float32),
    ],
)


@jax.jit
def kernel(x, edge_index, edge_weight, W_rel, b_rel, W_root, W_mu, b_mu,
           W_std, b_std):
  partials = _sc_scatter(x, edge_index.astype(jnp.int32), edge_weight)
  wcat = jnp.concatenate([W_mu.T, W_std.T], axis=1)
  bcat = jnp.concatenate([b_mu, b_std]).reshape(1, 2 * D)
  mu, ls = _dense(
      x, partials,
      W_rel.T, b_rel.reshape(1, D), W_root.T, wcat, bcat,
  )
  return (mu, ls)
